# baseline (speedup 1.0000x reference)
import functools

import jax
import jax.numpy as jnp
from jax import lax
from jax.experimental import pallas as pl
from jax.experimental.pallas import tpu as pltpu
from jax.experimental.pallas import tpu_sc as plsc


def _make_transpose_kernel(V, D, num_cores):
    ntc = (V + 127) // 128
    full = V // 128
    tail_v = V - full * 128
    nw = 32
    cpw = (ntc + nw - 1) // nw
    mesh = plsc.VectorSubcoreMesh(core_axis_name="c", subcore_axis_name="s")

    @functools.partial(
        pl.kernel,
        mesh=mesh,
        out_type=jax.ShapeDtypeStruct((V // 4, 4 * D), jnp.float32),
        scratch_types=[
            pltpu.VMEM((2, 4, 8, 128), jnp.float32),
            pltpu.VMEM((2, 32, 128), jnp.float32),
            pltpu.VMEM((16, 128), jnp.float32),
            pltpu.SemaphoreType.DMA,
            pltpu.SemaphoreType.DMA,
            pltpu.SemaphoreType.DMA,
            pltpu.SemaphoreType.DMA,
        ],
        compiler_params=pltpu.CompilerParams(
            use_tc_tiling_on_sc=True, needs_layout_passes=False
        ),
    )
    def tr(emb_t_hbm, tail16_hbm, trow_hbm, in_t, out_c, tail_c, g0, g1, o0, o1):
        wid = lax.axis_index("s") * num_cores + lax.axis_index("c")
        c_lo = wid * cpw
        c_hi = jnp.minimum(c_lo + cpw, full)
        n = jnp.maximum(c_hi - c_lo, 0)
        ii = lax.iota(jnp.int32, 16)

        def fetch_descs(c, slot):
            return [
                pltpu.make_async_copy(
                    emb_t_hbm.at[pl.ds(8 * k, 8), pl.ds(128 * c, 128)],
                    in_t.at[slot, k],
                    g0 if slot == 0 else g1,
                )
                for k in range(4)
            ]

        def fetch(c, slot):
            for d in fetch_descs(c, slot):
                d.start()

        def fetch_wait(c, slot):
            for d in fetch_descs(c, slot):
                d.wait()

        def out_desc(c, slot):
            return pltpu.make_async_copy(
                out_c.at[slot],
                trow_hbm.at[pl.ds(32 * c, 32), :],
                o0 if slot == 0 else o1,
            )

        def transpose_slot(slot):
            for j in range(0, 128, 16):
                iij = ii + j
                rvec = iij >> 2
                cbase = (iij & 3) * 32
                for d in range(D):
                    k, r = d >> 3, d & 7
                    v = in_t[slot, k, r, pl.ds(j, 16)]
                    plsc.store_scatter(out_c.at[slot], [rvec, cbase + d], v)

        @pl.when(n > 0)
        def _():
            fetch(c_lo, 0)

            def body(i2, carry):
                c0 = c_lo + 2 * i2

                @pl.when(2 * i2 < n)
                def _():
                    fetch_wait(c0, 0)

                    @pl.when(2 * i2 + 1 < n)
                    def _():
                        fetch(c0 + 1, 1)

                    @pl.when(i2 >= 1)
                    def _():
                        out_desc(c0 - 2, 0).wait()

                    transpose_slot(0)
                    out_desc(c0, 0).start()

                @pl.when(2 * i2 + 1 < n)
                def _():
                    fetch_wait(c0 + 1, 1)

                    @pl.when(2 * i2 + 2 < n)
                    def _():
                        fetch(c0 + 2, 0)

                    @pl.when(i2 >= 1)
                    def _():
                        out_desc(c0 - 1, 1).wait()

                    transpose_slot(1)
                    out_desc(c0 + 1, 1).start()

                return carry

            nhalf = (cpw + 1) // 2
            lax.fori_loop(0, nhalf, body, 0)

            @pl.when(n % 2 == 1)
            def _():
                @pl.when(n >= 2)
                def _():
                    out_desc(c_hi - 2, 1).wait()

                out_desc(c_hi - 1, 0).wait()

            @pl.when(n % 2 == 0)
            def _():
                out_desc(c_hi - 2, 0).wait()
                out_desc(c_hi - 1, 1).wait()

        @pl.when(wid == nw - 1)
        def _():
            pltpu.sync_copy(tail16_hbm, tail_c)
            pltpu.sync_copy(tail_c, trow_hbm.at[pl.ds(32 * full, 16), :])

    return tr


def _make_gather_kernel(N, D, n_per_w, chunk, num_cores):
    n_ch = n_per_w // chunk
    assert n_ch % 2 == 0 and n_ch * chunk == n_per_w
    mesh = plsc.VectorSubcoreMesh(core_axis_name="c", subcore_axis_name="s")

    @functools.partial(
        pl.kernel,
        mesh=mesh,
        out_type=jax.ShapeDtypeStruct((N, D), jnp.float32),
        scratch_types=[
            pltpu.VMEM((n_per_w,), jnp.int32),
            pltpu.VMEM((chunk, D), jnp.float32),
            pltpu.VMEM((chunk, D), jnp.float32),
            pltpu.SemaphoreType.DMA,
            pltpu.SemaphoreType.DMA,
            pltpu.SemaphoreType.DMA,
            pltpu.SemaphoreType.DMA,
        ],
        compiler_params=pltpu.CompilerParams(use_tc_tiling_on_sc=False),
    )
    def emb(idx_hbm, table_hbm, out_hbm, idx_v, r0, r1, g0, g1, o0, o1):
        wid = lax.axis_index("s") * num_cores + lax.axis_index("c")
        base = wid * n_per_w

        pltpu.sync_copy(idx_hbm.at[pl.ds(base, n_per_w)], idx_v)

        def gather(c, buf, sem):
            return pltpu.make_async_copy(
                table_hbm.at[idx_v.at[pl.ds(c * chunk, chunk)]], buf, sem
            )

        def outcp(c, buf, sem):
            return pltpu.make_async_copy(
                buf, out_hbm.at[pl.ds(base + c * chunk, chunk)], sem
            )

        gather(0, r0, g0).start()

        def body(i, carry):
            c0 = 2 * i
            c1 = 2 * i + 1
            gather(c0, r0, g0).wait()

            @pl.when(i >= 1)
            def _():
                outcp(c1 - 2, r1, o1).wait()

            gather(c1, r1, g1).start()
            outcp(c0, r0, o0).start()

            gather(c1, r1, g1).wait()
            outcp(c0, r0, o0).wait()

            @pl.when(i < (n_ch // 2 - 1))
            def _():
                gather(c0 + 2, r0, g0).start()

            outcp(c1, r1, o1).start()
            return carry

        lax.fori_loop(0, n_ch // 2, body, 0)
        outcp(n_ch - 1, r1, o1).wait()

    return emb


def kernel(x, embedding_matrix):
    B, H = x.shape
    V, D = embedding_matrix.shape
    N = B * H

    info = plsc.get_sparse_core_info()
    nw = info.num_cores * info.num_subcores
    n_per_w = N // nw
    chunk = 1600

    emb_t = jnp.transpose(embedding_matrix)
    full = V // 128
    tail16 = lax.slice(
        embedding_matrix, (128 * full, 0), (V, 0 + D)
    ).reshape((V - 128 * full) * D // 128, 128)
    tr = _make_transpose_kernel(V, D, info.num_cores)
    trow = tr(emb_t, tail16)
    table_lin = trow.reshape(V, D)

    idx = x.reshape(N).astype(jnp.int32)
    emb = _make_gather_kernel(N, D, n_per_w, chunk, info.num_cores)
    out = emb(idx, table_lin)
    return out.reshape(B, H, D)


# speedup vs baseline: 1.1491x; 1.1491x over previous
import functools

import jax
import jax.numpy as jnp
from jax import lax
from jax.experimental import pallas as pl
from jax.experimental.pallas import tpu as pltpu
from jax.experimental.pallas import tpu_sc as plsc


def _make_transpose_kernel(V, D, num_cores):
    ntc = (V + 127) // 128
    full = V // 128
    tail_v = V - full * 128
    nw = 32
    cpw = (ntc + nw - 1) // nw
    mesh = plsc.VectorSubcoreMesh(core_axis_name="c", subcore_axis_name="s")

    @functools.partial(
        pl.kernel,
        mesh=mesh,
        out_type=jax.ShapeDtypeStruct((V // 4, 4 * D), jnp.float32),
        scratch_types=[
            pltpu.VMEM((2, 4, 8, 128), jnp.float32),
            pltpu.VMEM((2, 32, 128), jnp.float32),
            pltpu.VMEM((16, 128), jnp.float32),
            pltpu.SemaphoreType.DMA,
            pltpu.SemaphoreType.DMA,
            pltpu.SemaphoreType.DMA,
            pltpu.SemaphoreType.DMA,
        ],
        compiler_params=pltpu.CompilerParams(
            use_tc_tiling_on_sc=True, needs_layout_passes=False
        ),
    )
    def tr(emb_t_hbm, tail16_hbm, trow_hbm, in_t, out_c, tail_c, g0, g1, o0, o1):
        wid = lax.axis_index("s") * num_cores + lax.axis_index("c")
        c_lo = wid * cpw
        c_hi = jnp.minimum(c_lo + cpw, full)
        n = jnp.maximum(c_hi - c_lo, 0)
        ii = lax.iota(jnp.int32, 16)

        def fetch_descs(c, slot):
            return [
                pltpu.make_async_copy(
                    emb_t_hbm.at[pl.ds(8 * k, 8), pl.ds(128 * c, 128)],
                    in_t.at[slot, k],
                    g0 if slot == 0 else g1,
                )
                for k in range(4)
            ]

        def fetch(c, slot):
            for d in fetch_descs(c, slot):
                d.start()

        def fetch_wait(c, slot):
            for d in fetch_descs(c, slot):
                d.wait()

        def out_desc(c, slot):
            return pltpu.make_async_copy(
                out_c.at[slot],
                trow_hbm.at[pl.ds(32 * c, 32), :],
                o0 if slot == 0 else o1,
            )

        def transpose_slot(slot):
            for j in range(0, 128, 16):
                iij = ii + j
                rvec = iij >> 2
                cbase = (iij & 3) * 32
                vals = [
                    in_t[slot, d >> 3, d & 7, pl.ds(j, 16)] for d in range(D)
                ]
                cols = [cbase + d for d in range(D)]
                for d in range(D):
                    plsc.store_scatter(out_c.at[slot], [rvec, cols[d]], vals[d])

        @pl.when(n > 0)
        def _():
            fetch(c_lo, 0)

            def body(i2, carry):
                c0 = c_lo + 2 * i2

                @pl.when(2 * i2 < n)
                def _():
                    fetch_wait(c0, 0)

                    @pl.when(2 * i2 + 1 < n)
                    def _():
                        fetch(c0 + 1, 1)

                    @pl.when(i2 >= 1)
                    def _():
                        out_desc(c0 - 2, 0).wait()

                    transpose_slot(0)
                    out_desc(c0, 0).start()

                @pl.when(2 * i2 + 1 < n)
                def _():
                    fetch_wait(c0 + 1, 1)

                    @pl.when(2 * i2 + 2 < n)
                    def _():
                        fetch(c0 + 2, 0)

                    @pl.when(i2 >= 1)
                    def _():
                        out_desc(c0 - 1, 1).wait()

                    transpose_slot(1)
                    out_desc(c0 + 1, 1).start()

                return carry

            nhalf = (cpw + 1) // 2
            lax.fori_loop(0, nhalf, body, 0)

            @pl.when(n % 2 == 1)
            def _():
                @pl.when(n >= 2)
                def _():
                    out_desc(c_hi - 2, 1).wait()

                out_desc(c_hi - 1, 0).wait()

            @pl.when(n % 2 == 0)
            def _():
                out_desc(c_hi - 2, 0).wait()
                out_desc(c_hi - 1, 1).wait()

        @pl.when(wid == nw - 1)
        def _():
            pltpu.sync_copy(tail16_hbm, tail_c)
            pltpu.sync_copy(tail_c, trow_hbm.at[pl.ds(32 * full, 16), :])

    return tr


def _make_gather_kernel(N, D, n_per_w, chunk, num_cores):
    n_ch = n_per_w // chunk
    assert n_ch % 2 == 0 and n_ch * chunk == n_per_w
    mesh = plsc.VectorSubcoreMesh(core_axis_name="c", subcore_axis_name="s")

    @functools.partial(
        pl.kernel,
        mesh=mesh,
        out_type=jax.ShapeDtypeStruct((N, D), jnp.float32),
        scratch_types=[
            pltpu.VMEM((n_per_w,), jnp.int32),
            pltpu.VMEM((chunk, D), jnp.float32),
            pltpu.VMEM((chunk, D), jnp.float32),
            pltpu.SemaphoreType.DMA,
            pltpu.SemaphoreType.DMA,
            pltpu.SemaphoreType.DMA,
            pltpu.SemaphoreType.DMA,
        ],
        compiler_params=pltpu.CompilerParams(use_tc_tiling_on_sc=False),
    )
    def emb(idx_hbm, table_hbm, out_hbm, idx_v, r0, r1, g0, g1, o0, o1):
        wid = lax.axis_index("s") * num_cores + lax.axis_index("c")
        base = wid * n_per_w

        pltpu.sync_copy(idx_hbm.at[pl.ds(base, n_per_w)], idx_v)

        def gather(c, buf, sem):
            return pltpu.make_async_copy(
                table_hbm.at[idx_v.at[pl.ds(c * chunk, chunk)]], buf, sem
            )

        def outcp(c, buf, sem):
            return pltpu.make_async_copy(
                buf, out_hbm.at[pl.ds(base + c * chunk, chunk)], sem
            )

        gather(0, r0, g0).start()

        def body(i, carry):
            c0 = 2 * i
            c1 = 2 * i + 1
            gather(c0, r0, g0).wait()

            @pl.when(i >= 1)
            def _():
                outcp(c1 - 2, r1, o1).wait()

            gather(c1, r1, g1).start()
            outcp(c0, r0, o0).start()

            gather(c1, r1, g1).wait()
            outcp(c0, r0, o0).wait()

            @pl.when(i < (n_ch // 2 - 1))
            def _():
                gather(c0 + 2, r0, g0).start()

            outcp(c1, r1, o1).start()
            return carry

        lax.fori_loop(0, n_ch // 2, body, 0)
        outcp(n_ch - 1, r1, o1).wait()

    return emb


def kernel(x, embedding_matrix):
    B, H = x.shape
    V, D = embedding_matrix.shape
    N = B * H

    info = plsc.get_sparse_core_info()
    nw = info.num_cores * info.num_subcores
    n_per_w = N // nw
    chunk = 1600

    emb_t = jnp.transpose(embedding_matrix)
    full = V // 128
    tail16 = lax.slice(
        embedding_matrix, (128 * full, 0), (V, 0 + D)
    ).reshape((V - 128 * full) * D // 128, 128)
    tr = _make_transpose_kernel(V, D, info.num_cores)
    trow = tr(emb_t, tail16)
    table_lin = trow.reshape(V, D)

    idx = x.reshape(N).astype(jnp.int32)
    emb = _make_gather_kernel(N, D, n_per_w, chunk, info.num_cores)
    out = emb(idx, table_lin)
    return out.reshape(B, H, D)


# speedup vs baseline: 1.3877x; 1.2076x over previous
import functools

import jax
import jax.numpy as jnp
from jax import lax
from jax.experimental import pallas as pl
from jax.experimental.pallas import tpu as pltpu
from jax.experimental.pallas import tpu_sc as plsc


def _make_transpose_kernel(V, D, num_cores):
    ntc = (V + 127) // 128
    full = V // 128
    tail_v = V - full * 128
    nw = 32
    cpw = (ntc + nw - 1) // nw
    mesh = plsc.VectorSubcoreMesh(core_axis_name="c", subcore_axis_name="s")

    @functools.partial(
        pl.kernel,
        mesh=mesh,
        out_type=jax.ShapeDtypeStruct((V // 4, 4 * D), jnp.float32),
        scratch_types=[
            pltpu.VMEM((2, 4, 8, 128), jnp.float32),
            pltpu.VMEM((128 * 33,), jnp.float32),
            pltpu.VMEM((2, 32, 128), jnp.float32),
            pltpu.VMEM((16, 128), jnp.float32),
            pltpu.SemaphoreType.DMA,
            pltpu.SemaphoreType.DMA,
            pltpu.SemaphoreType.DMA,
            pltpu.SemaphoreType.DMA,
        ],
        compiler_params=pltpu.CompilerParams(
            use_tc_tiling_on_sc=True, needs_layout_passes=False
        ),
    )
    def tr(
        emb_t_hbm, tail16_hbm, trow_hbm, in_t, stage, out_c, tail_c, g0, g1, o0, o1
    ):
        wid = lax.axis_index("s") * num_cores + lax.axis_index("c")
        c_lo = wid * cpw
        c_hi = jnp.minimum(c_lo + cpw, full)
        n = jnp.maximum(c_hi - c_lo, 0)
        ii = lax.iota(jnp.int32, 16)

        def fetch_descs(c, slot):
            return [
                pltpu.make_async_copy(
                    emb_t_hbm.at[pl.ds(8 * k, 8), pl.ds(128 * c, 128)],
                    in_t.at[slot, k],
                    g0 if slot == 0 else g1,
                )
                for k in range(4)
            ]

        def fetch(c, slot):
            for d in fetch_descs(c, slot):
                d.start()

        def fetch_wait(c, slot):
            for d in fetch_descs(c, slot):
                d.wait()

        def out_desc(c, slot):
            return pltpu.make_async_copy(
                out_c.at[slot],
                trow_hbm.at[pl.ds(32 * c, 32), :],
                o0 if slot == 0 else o1,
            )

        def transpose_slot(slot):
            for j in range(0, 128, 16):
                i33 = (ii + j) * 33
                vals = [
                    in_t[slot, d >> 3, d & 7, pl.ds(j, 16)] for d in range(D)
                ]
                for d in range(D):
                    plsc.store_scatter(stage, [i33 + d], vals[d])
            oc = out_c.at[slot]
            for i0 in range(0, 128, 4):
                vs = [
                    stage[pl.ds((i0 + q) * 33 + dh * 16, 16)]
                    for q in range(4)
                    for dh in range(2)
                ]
                for z in range(8):
                    q, dh = z // 2, z % 2
                    flat = (i0 + q) * 32 + dh * 16
                    oc[flat >> 7, pl.ds(flat & 127, 16)] = vs[z]

        @pl.when(n > 0)
        def _():
            fetch(c_lo, 0)

            def body(i2, carry):
                c0 = c_lo + 2 * i2

                @pl.when(2 * i2 < n)
                def _():
                    fetch_wait(c0, 0)

                    @pl.when(2 * i2 + 1 < n)
                    def _():
                        fetch(c0 + 1, 1)

                    @pl.when(i2 >= 1)
                    def _():
                        out_desc(c0 - 2, 0).wait()

                    transpose_slot(0)
                    out_desc(c0, 0).start()

                @pl.when(2 * i2 + 1 < n)
                def _():
                    fetch_wait(c0 + 1, 1)

                    @pl.when(2 * i2 + 2 < n)
                    def _():
                        fetch(c0 + 2, 0)

                    @pl.when(i2 >= 1)
                    def _():
                        out_desc(c0 - 1, 1).wait()

                    transpose_slot(1)
                    out_desc(c0 + 1, 1).start()

                return carry

            nhalf = (cpw + 1) // 2
            lax.fori_loop(0, nhalf, body, 0)

            @pl.when(n % 2 == 1)
            def _():
                @pl.when(n >= 2)
                def _():
                    out_desc(c_hi - 2, 1).wait()

                out_desc(c_hi - 1, 0).wait()

            @pl.when(n % 2 == 0)
            def _():
                out_desc(c_hi - 2, 0).wait()
                out_desc(c_hi - 1, 1).wait()

        @pl.when(wid == nw - 1)
        def _():
            pltpu.sync_copy(tail16_hbm, tail_c)
            pltpu.sync_copy(tail_c, trow_hbm.at[pl.ds(32 * full, 16), :])

    return tr


def _make_gather_kernel(N, D, n_per_w, chunk, num_cores):
    n_ch = n_per_w // chunk
    assert n_ch % 2 == 0 and n_ch * chunk == n_per_w
    mesh = plsc.VectorSubcoreMesh(core_axis_name="c", subcore_axis_name="s")

    @functools.partial(
        pl.kernel,
        mesh=mesh,
        out_type=jax.ShapeDtypeStruct((N, D), jnp.float32),
        scratch_types=[
            pltpu.VMEM((n_per_w,), jnp.int32),
            pltpu.VMEM((chunk, D), jnp.float32),
            pltpu.VMEM((chunk, D), jnp.float32),
            pltpu.SemaphoreType.DMA,
            pltpu.SemaphoreType.DMA,
            pltpu.SemaphoreType.DMA,
            pltpu.SemaphoreType.DMA,
        ],
        compiler_params=pltpu.CompilerParams(use_tc_tiling_on_sc=False),
    )
    def emb(idx_hbm, table_hbm, out_hbm, idx_v, r0, r1, g0, g1, o0, o1):
        wid = lax.axis_index("s") * num_cores + lax.axis_index("c")
        base = wid * n_per_w

        pltpu.sync_copy(idx_hbm.at[pl.ds(base, n_per_w)], idx_v)

        def gather(c, buf, sem):
            return pltpu.make_async_copy(
                table_hbm.at[idx_v.at[pl.ds(c * chunk, chunk)]], buf, sem
            )

        def outcp(c, buf, sem):
            return pltpu.make_async_copy(
                buf, out_hbm.at[pl.ds(base + c * chunk, chunk)], sem
            )

        gather(0, r0, g0).start()

        def body(i, carry):
            c0 = 2 * i
            c1 = 2 * i + 1
            gather(c0, r0, g0).wait()

            @pl.when(i >= 1)
            def _():
                outcp(c1 - 2, r1, o1).wait()

            gather(c1, r1, g1).start()
            outcp(c0, r0, o0).start()

            gather(c1, r1, g1).wait()
            outcp(c0, r0, o0).wait()

            @pl.when(i < (n_ch // 2 - 1))
            def _():
                gather(c0 + 2, r0, g0).start()

            outcp(c1, r1, o1).start()
            return carry

        lax.fori_loop(0, n_ch // 2, body, 0)
        outcp(n_ch - 1, r1, o1).wait()

    return emb


def kernel(x, embedding_matrix):
    B, H = x.shape
    V, D = embedding_matrix.shape
    N = B * H

    info = plsc.get_sparse_core_info()
    nw = info.num_cores * info.num_subcores
    n_per_w = N // nw
    chunk = 1600

    emb_t = jnp.transpose(embedding_matrix)
    full = V // 128
    tail16 = lax.slice(
        embedding_matrix, (128 * full, 0), (V, 0 + D)
    ).reshape((V - 128 * full) * D // 128, 128)
    tr = _make_transpose_kernel(V, D, info.num_cores)
    trow = tr(emb_t, tail16)
    table_lin = trow.reshape(V, D)

    idx = x.reshape(N).astype(jnp.int32)
    emb = _make_gather_kernel(N, D, n_per_w, chunk, info.num_cores)
    out = emb(idx, table_lin)
    return out.reshape(B, H, D)


# speedup vs baseline: 2.1754x; 1.5676x over previous
import functools

import jax
import jax.numpy as jnp
from jax import lax
from jax.experimental import pallas as pl
from jax.experimental.pallas import tpu as pltpu
from jax.experimental.pallas import tpu_sc as plsc


def _make_transpose_kernel(V, D, num_cores):
    ntc = (V + 127) // 128
    full = V // 128
    tail_v = V - full * 128
    nw = 32
    cpw = (ntc + nw - 1) // nw
    mesh = plsc.VectorSubcoreMesh(core_axis_name="c", subcore_axis_name="s")

    @functools.partial(
        pl.kernel,
        mesh=mesh,
        out_type=jax.ShapeDtypeStruct((V // 4, 4 * D), jnp.float32),
        scratch_types=[
            pltpu.VMEM((2, 4, 8, 128), jnp.float32),
            pltpu.VMEM((128 * 33,), jnp.float32),
            pltpu.VMEM((2, 32, 128), jnp.float32),
            pltpu.VMEM((16, 128), jnp.float32),
            pltpu.SemaphoreType.DMA,
            pltpu.SemaphoreType.DMA,
            pltpu.SemaphoreType.DMA,
            pltpu.SemaphoreType.DMA,
        ],
        compiler_params=pltpu.CompilerParams(
            use_tc_tiling_on_sc=True, needs_layout_passes=False
        ),
    )
    def tr(
        emb_t_hbm, tail16_hbm, trow_hbm, in_t, stage, out_c, tail_c, g0, g1, o0, o1
    ):
        wid = lax.axis_index("s") * num_cores + lax.axis_index("c")
        c_lo = wid * cpw
        c_hi = jnp.minimum(c_lo + cpw, full)
        n = jnp.maximum(c_hi - c_lo, 0)
        ii = lax.iota(jnp.int32, 16)

        def fetch_descs(c, slot):
            return [
                pltpu.make_async_copy(
                    emb_t_hbm.at[pl.ds(8 * k, 8), pl.ds(128 * c, 128)],
                    in_t.at[slot, k],
                    g0 if slot == 0 else g1,
                )
                for k in range(4)
            ]

        def fetch(c, slot):
            for d in fetch_descs(c, slot):
                d.start()

        def fetch_wait(c, slot):
            for d in fetch_descs(c, slot):
                d.wait()

        def out_desc(c, slot):
            return pltpu.make_async_copy(
                out_c.at[slot],
                trow_hbm.at[pl.ds(32 * c, 32), :],
                o0 if slot == 0 else o1,
            )

        def transpose_slot(slot):
            for j in range(0, 128, 16):
                i33 = (ii + j) * 33
                vals = [
                    in_t[slot, d >> 3, d & 7, pl.ds(j, 16)] for d in range(D)
                ]
                for d in range(D):
                    plsc.store_scatter(stage, [i33 + d], vals[d])
            oc = out_c.at[slot]
            for i0 in range(0, 128, 4):
                vs = [
                    stage[pl.ds((i0 + q) * 33 + dh * 16, 16)]
                    for q in range(4)
                    for dh in range(2)
                ]
                for z in range(8):
                    q, dh = z // 2, z % 2
                    flat = (i0 + q) * 32 + dh * 16
                    oc[flat >> 7, pl.ds(flat & 127, 16)] = vs[z]

        @pl.when(n > 0)
        def _():
            fetch(c_lo, 0)

            def body(i2, carry):
                c0 = c_lo + 2 * i2

                @pl.when(2 * i2 < n)
                def _():
                    fetch_wait(c0, 0)

                    @pl.when(2 * i2 + 1 < n)
                    def _():
                        fetch(c0 + 1, 1)

                    @pl.when(i2 >= 1)
                    def _():
                        out_desc(c0 - 2, 0).wait()

                    transpose_slot(0)
                    out_desc(c0, 0).start()

                @pl.when(2 * i2 + 1 < n)
                def _():
                    fetch_wait(c0 + 1, 1)

                    @pl.when(2 * i2 + 2 < n)
                    def _():
                        fetch(c0 + 2, 0)

                    @pl.when(i2 >= 1)
                    def _():
                        out_desc(c0 - 1, 1).wait()

                    transpose_slot(1)
                    out_desc(c0 + 1, 1).start()

                return carry

            nhalf = (cpw + 1) // 2
            lax.fori_loop(0, nhalf, body, 0)

            @pl.when(n % 2 == 1)
            def _():
                @pl.when(n >= 2)
                def _():
                    out_desc(c_hi - 2, 1).wait()

                out_desc(c_hi - 1, 0).wait()

            @pl.when(n % 2 == 0)
            def _():
                out_desc(c_hi - 2, 0).wait()
                out_desc(c_hi - 1, 1).wait()

        @pl.when(wid == nw - 1)
        def _():
            pltpu.sync_copy(tail16_hbm, tail_c)
            pltpu.sync_copy(tail_c, trow_hbm.at[pl.ds(32 * full, 16), :])

    return tr


def _make_gather_kernel(B, H, D, num_cores):
    hb = (H + 15) // 16
    mesh = plsc.VectorSubcoreMesh(core_axis_name="c", subcore_axis_name="s")

    @functools.partial(
        pl.kernel,
        mesh=mesh,
        out_type=jax.ShapeDtypeStruct((H, D // 8, B // 128, 8, 128), jnp.float32),
        scratch_types=[
            pltpu.VMEM((128 * H,), jnp.int32),
            pltpu.VMEM((hb * 16 * 129,), jnp.int32),
            pltpu.VMEM((H, 128), jnp.int32),
            pltpu.VMEM((2, 128, D), jnp.float32),
            pltpu.VMEM((128 * 33,), jnp.float32),
            pltpu.VMEM((2, D // 8, 8, 128), jnp.float32),
            pltpu.SemaphoreType.DMA,
            pltpu.SemaphoreType.DMA,
            pltpu.SemaphoreType.DMA,
            pltpu.SemaphoreType.DMA,
        ],
        compiler_params=pltpu.CompilerParams(
            use_tc_tiling_on_sc=False, needs_layout_passes=False
        ),
    )
    def emb(
        idx_hbm, table_hbm, out_hbm, idx_v, idx_s, idx_t, rows, stage, tv,
        g0, g1, o0, o1,
    ):
        wid = lax.axis_index("s") * num_cores + lax.axis_index("c")
        bt = wid
        ii = lax.iota(jnp.int32, 16)

        pltpu.sync_copy(idx_hbm.at[pl.ds(bt * 128 * H, 128 * H)], idx_v)

        def idx_tr_a(j, carry):
            for h0 in range(0, H, 16):
                v = idx_v[pl.ds(j * H + h0, 16)]
                plsc.store_scatter(idx_s, [(ii + h0) * 129 + j], v)
            return carry

        lax.fori_loop(0, 128, idx_tr_a, 0)

        def idx_tr_b(h, carry):
            for j0 in range(0, 128, 16):
                idx_t[h, pl.ds(j0, 16)] = idx_s[pl.ds(h * 129 + j0, 16)]
            return carry

        lax.fori_loop(0, H, idx_tr_b, 0)

        def gather(h, slot):
            return pltpu.make_async_copy(
                table_hbm.at[idx_t.at[h]],
                rows.at[slot],
                g0 if slot == 0 else g1,
            )

        def outcp(h, slot):
            return pltpu.make_async_copy(
                tv.at[slot],
                out_hbm.at[h, :, bt],
                o0 if slot == 0 else o1,
            )

        i129 = ii * 129

        def transpose_rows2(slot):
            def st_a(i, carry):
                j0 = i * 16
                vals = [
                    rows[slot, j0 + t, pl.ds(dh * 16, 16)]
                    for t in range(16)
                    for dh in range(2)
                ]
                z = 0
                for t in range(16):
                    for dh in range(2):
                        plsc.store_scatter(
                            stage, [i129 + (2064 * dh + j0 + t)], vals[z]
                        )
                        z += 1
                return carry

            lax.fori_loop(0, 8, st_a, 0)

            def st_b(i, carry):
                j0 = i * 16
                for d in range(D):
                    tv[slot, d >> 3, d & 7, pl.ds(j0, 16)] = stage[
                        pl.ds(d * 129 + j0, 16)
                    ]
                return carry

            lax.fori_loop(0, 8, st_b, 0)

        gather(0, 0).start()

        def body(i, carry):
            h0 = 2 * i
            h1 = 2 * i + 1
            gather(h0, 0).wait()

            @pl.when(h1 < H)
            def _():
                gather(h1, 1).start()

            @pl.when(i >= 1)
            def _():
                outcp(h0 - 2, 0).wait()

            transpose_rows2(0)
            outcp(h0, 0).start()

            @pl.when(h1 < H)
            def _():
                gather(h1, 1).wait()

                @pl.when(h1 + 1 < H)
                def _():
                    gather(h1 + 1, 0).start()

                @pl.when(i >= 1)
                def _():
                    outcp(h1 - 2, 1).wait()

                transpose_rows2(1)
                outcp(h1, 1).start()

            return carry

        lax.fori_loop(0, (H + 1) // 2, body, 0)
        outcp(H - 2, 0).wait()
        outcp(H - 1, 1).wait()

    return emb


def kernel(x, embedding_matrix):
    B, H = x.shape
    V, D = embedding_matrix.shape
    N = B * H

    info = plsc.get_sparse_core_info()
    nw = info.num_cores * info.num_subcores
    n_per_w = N // nw
    chunk = 1600

    emb_t = jnp.transpose(embedding_matrix)
    full = V // 128
    tail16 = lax.slice(
        embedding_matrix, (128 * full, 0), (V, 0 + D)
    ).reshape((V - 128 * full) * D // 128, 128)
    tr = _make_transpose_kernel(V, D, info.num_cores)
    trow = tr(emb_t, tail16)
    table_lin = trow.reshape(V, D)

    idx = x.reshape(N).astype(jnp.int32)
    emb = _make_gather_kernel(B, H, D, info.num_cores)
    p = emb(idx, table_lin)
    return p.transpose(2, 4, 0, 1, 3).reshape(B, H, D)


# speedup vs baseline: 2.4976x; 1.1481x over previous
import functools

import jax
import jax.numpy as jnp
from jax import lax
from jax.experimental import pallas as pl
from jax.experimental.pallas import tpu as pltpu
from jax.experimental.pallas import tpu_sc as plsc


def _make_transpose_kernel(V, D, num_cores):
    ntc = (V + 127) // 128
    full = V // 128
    tail_v = V - full * 128
    nw = 32
    cpw = (ntc + nw - 1) // nw
    mesh = plsc.VectorSubcoreMesh(core_axis_name="c", subcore_axis_name="s")

    @functools.partial(
        pl.kernel,
        mesh=mesh,
        out_type=jax.ShapeDtypeStruct((V // 4, 4 * D), jnp.float32),
        scratch_types=[
            pltpu.VMEM((2, 4, 8, 128), jnp.float32),
            pltpu.VMEM((128 * 33,), jnp.float32),
            pltpu.VMEM((2, 32, 128), jnp.float32),
            pltpu.VMEM((16, 128), jnp.float32),
            pltpu.SemaphoreType.DMA,
            pltpu.SemaphoreType.DMA,
            pltpu.SemaphoreType.DMA,
            pltpu.SemaphoreType.DMA,
        ],
        compiler_params=pltpu.CompilerParams(
            use_tc_tiling_on_sc=True, needs_layout_passes=False
        ),
    )
    def tr(
        emb_t_hbm, tail16_hbm, trow_hbm, in_t, stage, out_c, tail_c, g0, g1, o0, o1
    ):
        wid = lax.axis_index("s") * num_cores + lax.axis_index("c")
        c_lo = wid * cpw
        c_hi = jnp.minimum(c_lo + cpw, full)
        n = jnp.maximum(c_hi - c_lo, 0)
        ii = lax.iota(jnp.int32, 16)

        def fetch_descs(c, slot):
            return [
                pltpu.make_async_copy(
                    emb_t_hbm.at[pl.ds(8 * k, 8), pl.ds(128 * c, 128)],
                    in_t.at[slot, k],
                    g0 if slot == 0 else g1,
                )
                for k in range(4)
            ]

        def fetch(c, slot):
            for d in fetch_descs(c, slot):
                d.start()

        def fetch_wait(c, slot):
            for d in fetch_descs(c, slot):
                d.wait()

        def out_desc(c, slot):
            return pltpu.make_async_copy(
                out_c.at[slot],
                trow_hbm.at[pl.ds(32 * c, 32), :],
                o0 if slot == 0 else o1,
            )

        def transpose_slot(slot):
            @plsc.parallel_loop(0, 128, 16, unroll=2)
            def _(j):
                i33 = (ii + j) * 33
                vals = [
                    in_t[slot, d >> 3, d & 7, pl.ds(j, 16)] for d in range(D)
                ]
                for d in range(D):
                    plsc.store_scatter(stage, [i33 + d], vals[d])

            oc = out_c.at[slot]

            @plsc.parallel_loop(0, 32, 1, unroll=4)
            def _(r):
                base_r = r * 4 * 33
                for z in range(8):
                    q, dh = z // 2, z % 2
                    oc[r, pl.ds(q * 32 + dh * 16, 16)] = stage[
                        pl.ds(base_r + q * 33 + dh * 16, 16)
                    ]

        @pl.when(n > 0)
        def _():
            fetch(c_lo, 0)

            def body(i2, carry):
                c0 = c_lo + 2 * i2

                @pl.when(2 * i2 < n)
                def _():
                    fetch_wait(c0, 0)

                    @pl.when(2 * i2 + 1 < n)
                    def _():
                        fetch(c0 + 1, 1)

                    @pl.when(i2 >= 1)
                    def _():
                        out_desc(c0 - 2, 0).wait()

                    transpose_slot(0)
                    out_desc(c0, 0).start()

                @pl.when(2 * i2 + 1 < n)
                def _():
                    fetch_wait(c0 + 1, 1)

                    @pl.when(2 * i2 + 2 < n)
                    def _():
                        fetch(c0 + 2, 0)

                    @pl.when(i2 >= 1)
                    def _():
                        out_desc(c0 - 1, 1).wait()

                    transpose_slot(1)
                    out_desc(c0 + 1, 1).start()

                return carry

            nhalf = (cpw + 1) // 2
            lax.fori_loop(0, nhalf, body, 0)

            @pl.when(n % 2 == 1)
            def _():
                @pl.when(n >= 2)
                def _():
                    out_desc(c_hi - 2, 1).wait()

                out_desc(c_hi - 1, 0).wait()

            @pl.when(n % 2 == 0)
            def _():
                out_desc(c_hi - 2, 0).wait()
                out_desc(c_hi - 1, 1).wait()

        @pl.when(wid == nw - 1)
        def _():
            pltpu.sync_copy(tail16_hbm, tail_c)
            pltpu.sync_copy(tail_c, trow_hbm.at[pl.ds(32 * full, 16), :])

    return tr


def _make_gather_kernel(B, H, D, num_cores):
    hb = (H + 15) // 16
    mesh = plsc.VectorSubcoreMesh(core_axis_name="c", subcore_axis_name="s")

    @functools.partial(
        pl.kernel,
        mesh=mesh,
        out_type=jax.ShapeDtypeStruct((H, D // 8, B // 128, 8, 128), jnp.float32),
        scratch_types=[
            pltpu.VMEM((128 * H,), jnp.int32),
            pltpu.VMEM((hb * 16 * 129,), jnp.int32),
            pltpu.VMEM((H, 128), jnp.int32),
            pltpu.VMEM((2, 128, D), jnp.float32),
            pltpu.VMEM((128 * 33,), jnp.float32),
            pltpu.VMEM((2, D // 8, 8, 128), jnp.float32),
            pltpu.SemaphoreType.DMA,
            pltpu.SemaphoreType.DMA,
            pltpu.SemaphoreType.DMA,
            pltpu.SemaphoreType.DMA,
        ],
        compiler_params=pltpu.CompilerParams(
            use_tc_tiling_on_sc=False, needs_layout_passes=False
        ),
    )
    def emb(
        idx_hbm, table_hbm, out_hbm, idx_v, idx_s, idx_t, rows, stage, tv,
        g0, g1, o0, o1,
    ):
        wid = lax.axis_index("s") * num_cores + lax.axis_index("c")
        bt = wid
        ii = lax.iota(jnp.int32, 16)

        pltpu.sync_copy(idx_hbm.at[pl.ds(bt * 128 * H, 128 * H)], idx_v)

        def idx_tr_a(j, carry):
            for h0 in range(0, H, 16):
                v = idx_v[pl.ds(j * H + h0, 16)]
                plsc.store_scatter(idx_s, [(ii + h0) * 129 + j], v)
            return carry

        lax.fori_loop(0, 128, idx_tr_a, 0)

        def idx_tr_b(h, carry):
            for j0 in range(0, 128, 16):
                idx_t[h, pl.ds(j0, 16)] = idx_s[pl.ds(h * 129 + j0, 16)]
            return carry

        lax.fori_loop(0, H, idx_tr_b, 0)

        def gather(h, slot):
            return pltpu.make_async_copy(
                table_hbm.at[idx_t.at[h]],
                rows.at[slot],
                g0 if slot == 0 else g1,
            )

        def outcp(h, slot):
            return pltpu.make_async_copy(
                tv.at[slot],
                out_hbm.at[h, :, bt],
                o0 if slot == 0 else o1,
            )

        i129 = ii * 129

        def transpose_rows2(slot):
            def st_a(i, carry):
                j0 = i * 16
                vals = [
                    rows[slot, j0 + t, pl.ds(dh * 16, 16)]
                    for t in range(16)
                    for dh in range(2)
                ]
                z = 0
                for t in range(16):
                    for dh in range(2):
                        plsc.store_scatter(
                            stage, [i129 + (2064 * dh + j0 + t)], vals[z]
                        )
                        z += 1
                return carry

            lax.fori_loop(0, 8, st_a, 0)

            def st_b(i, carry):
                j0 = i * 16
                for d in range(D):
                    tv[slot, d >> 3, d & 7, pl.ds(j0, 16)] = stage[
                        pl.ds(d * 129 + j0, 16)
                    ]
                return carry

            lax.fori_loop(0, 8, st_b, 0)

        gather(0, 0).start()

        def body(i, carry):
            h0 = 2 * i
            h1 = 2 * i + 1
            gather(h0, 0).wait()

            @pl.when(h1 < H)
            def _():
                gather(h1, 1).start()

            @pl.when(i >= 1)
            def _():
                outcp(h0 - 2, 0).wait()

            transpose_rows2(0)
            outcp(h0, 0).start()

            @pl.when(h1 < H)
            def _():
                gather(h1, 1).wait()

                @pl.when(h1 + 1 < H)
                def _():
                    gather(h1 + 1, 0).start()

                @pl.when(i >= 1)
                def _():
                    outcp(h1 - 2, 1).wait()

                transpose_rows2(1)
                outcp(h1, 1).start()

            return carry

        lax.fori_loop(0, (H + 1) // 2, body, 0)
        outcp(H - 2, 0).wait()
        outcp(H - 1, 1).wait()

    return emb


def kernel(x, embedding_matrix):
    B, H = x.shape
    V, D = embedding_matrix.shape
    N = B * H

    info = plsc.get_sparse_core_info()
    nw = info.num_cores * info.num_subcores
    n_per_w = N // nw
    chunk = 1600

    emb_t = jnp.transpose(embedding_matrix)
    full = V // 128
    tail16 = lax.slice(
        embedding_matrix, (128 * full, 0), (V, 0 + D)
    ).reshape((V - 128 * full) * D // 128, 128)
    tr = _make_transpose_kernel(V, D, info.num_cores)
    trow = tr(emb_t, tail16)
    table_lin = trow.reshape(V, D)

    idx = x.reshape(N).astype(jnp.int32)
    emb = _make_gather_kernel(B, H, D, info.num_cores)
    p = emb(idx, table_lin)
    return p.transpose(2, 4, 0, 1, 3).reshape(B, H, D)


# speedup vs baseline: 2.5339x; 1.0145x over previous
import functools

import jax
import jax.numpy as jnp
from jax import lax
from jax.experimental import pallas as pl
from jax.experimental.pallas import tpu as pltpu
from jax.experimental.pallas import tpu_sc as plsc


def _make_transpose_kernel(V, D, num_cores):
    ntc = (V + 127) // 128
    full = V // 128
    tail_v = V - full * 128
    nw = 32
    cpw = (ntc + nw - 1) // nw
    mesh = plsc.VectorSubcoreMesh(core_axis_name="c", subcore_axis_name="s")

    @functools.partial(
        pl.kernel,
        mesh=mesh,
        out_type=jax.ShapeDtypeStruct((V // 4, 4 * D), jnp.float32),
        scratch_types=[
            pltpu.VMEM((2, 4, 8, 128), jnp.float32),
            pltpu.VMEM((128 * 33,), jnp.float32),
            pltpu.VMEM((2, 32, 128), jnp.float32),
            pltpu.VMEM((16, 128), jnp.float32),
            pltpu.SemaphoreType.DMA,
            pltpu.SemaphoreType.DMA,
            pltpu.SemaphoreType.DMA,
            pltpu.SemaphoreType.DMA,
        ],
        compiler_params=pltpu.CompilerParams(
            use_tc_tiling_on_sc=True, needs_layout_passes=False
        ),
    )
    def tr(
        emb_t_hbm, tail16_hbm, trow_hbm, in_t, stage, out_c, tail_c, g0, g1, o0, o1
    ):
        wid = lax.axis_index("s") * num_cores + lax.axis_index("c")
        c_lo = wid * cpw
        c_hi = jnp.minimum(c_lo + cpw, full)
        n = jnp.maximum(c_hi - c_lo, 0)
        ii = lax.iota(jnp.int32, 16)

        def fetch_descs(c, slot):
            return [
                pltpu.make_async_copy(
                    emb_t_hbm.at[pl.ds(8 * k, 8), pl.ds(128 * c, 128)],
                    in_t.at[slot, k],
                    g0 if slot == 0 else g1,
                )
                for k in range(4)
            ]

        def fetch(c, slot):
            for d in fetch_descs(c, slot):
                d.start()

        def fetch_wait(c, slot):
            for d in fetch_descs(c, slot):
                d.wait()

        def out_desc(c, slot):
            return pltpu.make_async_copy(
                out_c.at[slot],
                trow_hbm.at[pl.ds(32 * c, 32), :],
                o0 if slot == 0 else o1,
            )

        def transpose_slot(slot):
            @plsc.parallel_loop(0, 128, 16, unroll=2)
            def _(j):
                i33 = (ii + j) * 33
                vals = [
                    in_t[slot, d >> 3, d & 7, pl.ds(j, 16)] for d in range(D)
                ]
                for d in range(D):
                    plsc.store_scatter(stage, [i33 + d], vals[d])

            oc = out_c.at[slot]

            @plsc.parallel_loop(0, 32, 1, unroll=4)
            def _(r):
                base_r = r * 4 * 33
                for z in range(8):
                    q, dh = z // 2, z % 2
                    oc[r, pl.ds(q * 32 + dh * 16, 16)] = stage[
                        pl.ds(base_r + q * 33 + dh * 16, 16)
                    ]

        @pl.when(n > 0)
        def _():
            fetch(c_lo, 0)

            def body(i2, carry):
                c0 = c_lo + 2 * i2

                @pl.when(2 * i2 < n)
                def _():
                    fetch_wait(c0, 0)

                    @pl.when(2 * i2 + 1 < n)
                    def _():
                        fetch(c0 + 1, 1)

                    @pl.when(i2 >= 1)
                    def _():
                        out_desc(c0 - 2, 0).wait()

                    transpose_slot(0)
                    out_desc(c0, 0).start()

                @pl.when(2 * i2 + 1 < n)
                def _():
                    fetch_wait(c0 + 1, 1)

                    @pl.when(2 * i2 + 2 < n)
                    def _():
                        fetch(c0 + 2, 0)

                    @pl.when(i2 >= 1)
                    def _():
                        out_desc(c0 - 1, 1).wait()

                    transpose_slot(1)
                    out_desc(c0 + 1, 1).start()

                return carry

            nhalf = (cpw + 1) // 2
            lax.fori_loop(0, nhalf, body, 0)

            @pl.when(n % 2 == 1)
            def _():
                @pl.when(n >= 2)
                def _():
                    out_desc(c_hi - 2, 1).wait()

                out_desc(c_hi - 1, 0).wait()

            @pl.when(n % 2 == 0)
            def _():
                out_desc(c_hi - 2, 0).wait()
                out_desc(c_hi - 1, 1).wait()

        @pl.when(wid == nw - 1)
        def _():
            pltpu.sync_copy(tail16_hbm, tail_c)
            pltpu.sync_copy(tail_c, trow_hbm.at[pl.ds(32 * full, 16), :])

    return tr


def _make_gather_kernel(B, H, D, num_cores):
    hb = (H + 15) // 16
    mesh = plsc.VectorSubcoreMesh(core_axis_name="c", subcore_axis_name="s")

    @functools.partial(
        pl.kernel,
        mesh=mesh,
        out_type=jax.ShapeDtypeStruct((H, D // 8, B // 128, 8, 128), jnp.float32),
        scratch_types=[
            pltpu.VMEM((128 * H,), jnp.int32),
            pltpu.VMEM((hb * 16 * 129,), jnp.int32),
            pltpu.VMEM((H, 128), jnp.int32),
            pltpu.VMEM((2, 128, D), jnp.float32),
            pltpu.VMEM((128 * 33,), jnp.float32),
            pltpu.VMEM((2, D // 8, 8, 128), jnp.float32),
            pltpu.SemaphoreType.DMA,
            pltpu.SemaphoreType.DMA,
            pltpu.SemaphoreType.DMA,
            pltpu.SemaphoreType.DMA,
        ],
        compiler_params=pltpu.CompilerParams(
            use_tc_tiling_on_sc=False, needs_layout_passes=False
        ),
    )
    def emb(
        idx_hbm, table_hbm, out_hbm, idx_v, idx_s, idx_t, rows, stage, tv,
        g0, g1, o0, o1,
    ):
        wid = lax.axis_index("s") * num_cores + lax.axis_index("c")
        bt = wid
        ii = lax.iota(jnp.int32, 16)

        pltpu.sync_copy(idx_hbm.at[pl.ds(bt * 128 * H, 128 * H)], idx_v)

        @plsc.parallel_loop(0, 128, 1, unroll=2)
        def _(j):
            for h0 in range(0, H, 16):
                v = idx_v[pl.ds(j * H + h0, 16)]
                plsc.store_scatter(idx_s, [(ii + h0) * 129 + j], v)

        @plsc.parallel_loop(0, H, 1, unroll=2)
        def _(h):
            for j0 in range(0, 128, 16):
                idx_t[h, pl.ds(j0, 16)] = idx_s[pl.ds(h * 129 + j0, 16)]

        def gather(h, slot):
            return pltpu.make_async_copy(
                table_hbm.at[idx_t.at[h]],
                rows.at[slot],
                g0 if slot == 0 else g1,
            )

        def outcp(h, slot):
            return pltpu.make_async_copy(
                tv.at[slot],
                out_hbm.at[h, :, bt],
                o0 if slot == 0 else o1,
            )

        i129 = ii * 129

        def transpose_rows2(slot):
            @plsc.parallel_loop(0, 128, 16, unroll=2)
            def _(j0):
                vals = [
                    rows[slot, j0 + t, pl.ds(dh * 16, 16)]
                    for t in range(16)
                    for dh in range(2)
                ]
                z = 0
                for t in range(16):
                    for dh in range(2):
                        plsc.store_scatter(
                            stage, [i129 + (2064 * dh + j0 + t)], vals[z]
                        )
                        z += 1

            @plsc.parallel_loop(0, 128, 16, unroll=2)
            def _(j0):
                for d in range(D):
                    tv[slot, d >> 3, d & 7, pl.ds(j0, 16)] = stage[
                        pl.ds(d * 129 + j0, 16)
                    ]

        gather(0, 0).start()

        def body(i, carry):
            h0 = 2 * i
            h1 = 2 * i + 1
            gather(h0, 0).wait()

            @pl.when(h1 < H)
            def _():
                gather(h1, 1).start()

            @pl.when(i >= 1)
            def _():
                outcp(h0 - 2, 0).wait()

            transpose_rows2(0)
            outcp(h0, 0).start()

            @pl.when(h1 < H)
            def _():
                gather(h1, 1).wait()

                @pl.when(h1 + 1 < H)
                def _():
                    gather(h1 + 1, 0).start()

                @pl.when(i >= 1)
                def _():
                    outcp(h1 - 2, 1).wait()

                transpose_rows2(1)
                outcp(h1, 1).start()

            return carry

        lax.fori_loop(0, (H + 1) // 2, body, 0)
        outcp(H - 2, 0).wait()
        outcp(H - 1, 1).wait()

    return emb


def kernel(x, embedding_matrix):
    B, H = x.shape
    V, D = embedding_matrix.shape
    N = B * H

    info = plsc.get_sparse_core_info()
    nw = info.num_cores * info.num_subcores
    n_per_w = N // nw
    chunk = 1600

    emb_t = jnp.transpose(embedding_matrix)
    full = V // 128
    tail16 = lax.slice(
        embedding_matrix, (128 * full, 0), (V, 0 + D)
    ).reshape((V - 128 * full) * D // 128, 128)
    tr = _make_transpose_kernel(V, D, info.num_cores)
    trow = tr(emb_t, tail16)
    table_lin = trow.reshape(V, D)

    idx = x.reshape(N).astype(jnp.int32)
    emb = _make_gather_kernel(B, H, D, info.num_cores)
    p = emb(idx, table_lin)
    return p.transpose(2, 4, 0, 1, 3).reshape(B, H, D)


# speedup vs baseline: 2.5419x; 1.0032x over previous
import functools

import jax
import jax.numpy as jnp
from jax import lax
from jax.experimental import pallas as pl
from jax.experimental.pallas import tpu as pltpu
from jax.experimental.pallas import tpu_sc as plsc


def _make_transpose_kernel(V, D, num_cores):
    ntc = (V + 127) // 128
    full = V // 128
    tail_v = V - full * 128
    nw = 32
    cpw = (ntc + nw - 1) // nw
    mesh = plsc.VectorSubcoreMesh(core_axis_name="c", subcore_axis_name="s")

    @functools.partial(
        pl.kernel,
        mesh=mesh,
        out_type=jax.ShapeDtypeStruct((V // 4, 4 * D), jnp.float32),
        scratch_types=[
            pltpu.VMEM((2, 4, 8, 128), jnp.float32),
            pltpu.VMEM((128 * 33,), jnp.float32),
            pltpu.VMEM((2, 32, 128), jnp.float32),
            pltpu.VMEM((16, 128), jnp.float32),
            pltpu.SemaphoreType.DMA,
            pltpu.SemaphoreType.DMA,
            pltpu.SemaphoreType.DMA,
            pltpu.SemaphoreType.DMA,
        ],
        compiler_params=pltpu.CompilerParams(
            use_tc_tiling_on_sc=True, needs_layout_passes=False
        ),
    )
    def tr(
        emb_t_hbm, tail16_hbm, trow_hbm, in_t, stage, out_c, tail_c, g0, g1, o0, o1
    ):
        wid = lax.axis_index("s") * num_cores + lax.axis_index("c")
        c_lo = wid * cpw
        c_hi = jnp.minimum(c_lo + cpw, full)
        n = jnp.maximum(c_hi - c_lo, 0)
        ii = lax.iota(jnp.int32, 16)

        def fetch_descs(c, slot):
            return [
                pltpu.make_async_copy(
                    emb_t_hbm.at[pl.ds(8 * k, 8), pl.ds(128 * c, 128)],
                    in_t.at[slot, k],
                    g0 if slot == 0 else g1,
                )
                for k in range(4)
            ]

        def fetch(c, slot):
            for d in fetch_descs(c, slot):
                d.start()

        def fetch_wait(c, slot):
            for d in fetch_descs(c, slot):
                d.wait()

        def out_desc(c, slot):
            return pltpu.make_async_copy(
                out_c.at[slot],
                trow_hbm.at[pl.ds(32 * c, 32), :],
                o0 if slot == 0 else o1,
            )

        def transpose_slot(slot):
            @plsc.parallel_loop(0, 128, 16, unroll=2)
            def _(j):
                i33 = (ii + j) * 33
                vals = [
                    in_t[slot, d >> 3, d & 7, pl.ds(j, 16)] for d in range(D)
                ]
                for d in range(D):
                    plsc.store_scatter(stage, [i33 + d], vals[d])

            oc = out_c.at[slot]

            @plsc.parallel_loop(0, 32, 1, unroll=4)
            def _(r):
                base_r = r * 4 * 33
                for z in range(8):
                    q, dh = z // 2, z % 2
                    oc[r, pl.ds(q * 32 + dh * 16, 16)] = stage[
                        pl.ds(base_r + q * 33 + dh * 16, 16)
                    ]

        @pl.when(n > 0)
        def _():
            fetch(c_lo, 0)

            def body(i2, carry):
                c0 = c_lo + 2 * i2

                @pl.when(2 * i2 < n)
                def _():
                    fetch_wait(c0, 0)

                    @pl.when(2 * i2 + 1 < n)
                    def _():
                        fetch(c0 + 1, 1)

                    @pl.when(i2 >= 1)
                    def _():
                        out_desc(c0 - 2, 0).wait()

                    transpose_slot(0)
                    out_desc(c0, 0).start()

                @pl.when(2 * i2 + 1 < n)
                def _():
                    fetch_wait(c0 + 1, 1)

                    @pl.when(2 * i2 + 2 < n)
                    def _():
                        fetch(c0 + 2, 0)

                    @pl.when(i2 >= 1)
                    def _():
                        out_desc(c0 - 1, 1).wait()

                    transpose_slot(1)
                    out_desc(c0 + 1, 1).start()

                return carry

            nhalf = (cpw + 1) // 2
            lax.fori_loop(0, nhalf, body, 0)

            @pl.when(n % 2 == 1)
            def _():
                @pl.when(n >= 2)
                def _():
                    out_desc(c_hi - 2, 1).wait()

                out_desc(c_hi - 1, 0).wait()

            @pl.when(n % 2 == 0)
            def _():
                out_desc(c_hi - 2, 0).wait()
                out_desc(c_hi - 1, 1).wait()

        @pl.when(wid == nw - 1)
        def _():
            pltpu.sync_copy(tail16_hbm, tail_c)
            pltpu.sync_copy(tail_c, trow_hbm.at[pl.ds(32 * full, 16), :])

    return tr


def _make_gather_kernel(B, H, D, num_cores):
    hb = (H + 15) // 16
    mesh = plsc.VectorSubcoreMesh(core_axis_name="c", subcore_axis_name="s")

    @functools.partial(
        pl.kernel,
        mesh=mesh,
        out_type=jax.ShapeDtypeStruct((H, D // 8, B // 128, 8, 128), jnp.float32),
        scratch_types=[
            pltpu.VMEM((128 * H,), jnp.int32),
            pltpu.VMEM((hb * 16 * 129,), jnp.int32),
            pltpu.VMEM((H, 128), jnp.int32),
            pltpu.VMEM((2, 128, D), jnp.float32),
            pltpu.VMEM((128 * 33,), jnp.float32),
            pltpu.VMEM((2, D // 8, 8, 128), jnp.float32),
            pltpu.SemaphoreType.DMA,
            pltpu.SemaphoreType.DMA,
            pltpu.SemaphoreType.DMA,
            pltpu.SemaphoreType.DMA,
        ],
        compiler_params=pltpu.CompilerParams(
            use_tc_tiling_on_sc=False, needs_layout_passes=False
        ),
    )
    def emb(
        idx_hbm, table_hbm, out_hbm, idx_v, idx_s, idx_t, rows, stage, tv,
        g0, g1, o0, o1,
    ):
        wid = lax.axis_index("s") * num_cores + lax.axis_index("c")
        bt = wid
        ii = lax.iota(jnp.int32, 16)

        pltpu.sync_copy(idx_hbm.at[pl.ds(bt * 128 * H, 128 * H)], idx_v)

        @plsc.parallel_loop(0, 128, 1, unroll=2)
        def _(j):
            for h0 in range(0, H, 16):
                v = idx_v[pl.ds(j * H + h0, 16)]
                plsc.store_scatter(idx_s, [(ii + h0) * 129 + j], v)

        @plsc.parallel_loop(0, H, 1, unroll=2)
        def _(h):
            for j0 in range(0, 128, 16):
                idx_t[h, pl.ds(j0, 16)] = idx_s[pl.ds(h * 129 + j0, 16)]

        def gather(h, slot):
            return pltpu.make_async_copy(
                table_hbm.at[idx_t.at[h]],
                rows.at[slot],
                g0 if slot == 0 else g1,
            )

        def outcp(h, slot):
            return pltpu.make_async_copy(
                tv.at[slot],
                out_hbm.at[h, :, bt],
                o0 if slot == 0 else o1,
            )

        i129 = ii * 129

        def transpose_rows2(slot):
            @plsc.parallel_loop(0, 128, 16, unroll=2)
            def _(j0):
                vals = [
                    rows[slot, j0 + t, pl.ds(dh * 16, 16)]
                    for t in range(16)
                    for dh in range(2)
                ]
                z = 0
                for t in range(16):
                    for dh in range(2):
                        plsc.store_scatter(
                            stage, [i129 + (2064 * dh + j0 + t)], vals[z]
                        )
                        z += 1

            @plsc.parallel_loop(0, 128, 16, unroll=2)
            def _(j0):
                for d in range(D):
                    tv[slot, d >> 3, d & 7, pl.ds(j0, 16)] = stage[
                        pl.ds(d * 129 + j0, 16)
                    ]

        gather(0, 0).start()

        def body(i, carry):
            h0 = 2 * i
            h1 = 2 * i + 1
            gather(h0, 0).wait()

            @pl.when(h1 < H)
            def _():
                gather(h1, 1).start()

            @pl.when(i >= 1)
            def _():
                outcp(h0 - 2, 0).wait()

            transpose_rows2(0)
            outcp(h0, 0).start()

            @pl.when(h1 < H)
            def _():
                gather(h1, 1).wait()

                @pl.when(h1 + 1 < H)
                def _():
                    gather(h1 + 1, 0).start()

                @pl.when(i >= 1)
                def _():
                    outcp(h1 - 2, 1).wait()

                transpose_rows2(1)
                outcp(h1, 1).start()

            return carry

        lax.fori_loop(0, (H + 1) // 2, body, 0)
        outcp(H - 2, 0).wait()
        outcp(H - 1, 1).wait()

    return emb


def kernel(x, embedding_matrix):
    B, H = x.shape
    V, D = embedding_matrix.shape
    N = B * H

    info = plsc.get_sparse_core_info()

    emb_t = jnp.transpose(embedding_matrix)
    full = V // 128
    tail16 = lax.slice(
        embedding_matrix, (128 * full, 0), (V, 0 + D)
    ).reshape((V - 128 * full) * D // 128, 128)
    tr = _make_transpose_kernel(V, D, info.num_cores)
    trow = tr(emb_t, tail16)
    table_lin = trow.reshape(V, D)

    idx = x.reshape(N).astype(jnp.int32)
    emb = _make_gather_kernel(B, H, D, info.num_cores)
    p = emb(idx, table_lin)
    return p.transpose(2, 4, 0, 1, 3).reshape(B, H, D)
